# Initial kernel scaffold; baseline (speedup 1.0000x reference)
import jax
import jax.numpy as jnp
from jax.experimental import pallas as pl


def kernel(x, w1c, wb_all, wc_all, wm_all, res2w, wf, vecs):
    raise NotImplementedError("write your pallas kernel here")


# speedup vs baseline: 1.7662x; 1.7662x over previous
import jax
import jax.numpy as jnp
from jax import lax
from jax.experimental import pallas as pl
from jax.experimental.pallas import tpu as pltpu

L = 1600
PAD = 16
NTAP = 23
NCH = 128
NB = 32
NG = 3
GK = 8 * NB
YB = 19


def _row(vec_ref, r):
    return vec_ref[r:r + 1, :]


def _first_inception(x_ref, w1c_ref, vec_ref, acc_ref):
    acc_ref[...] = jnp.zeros((L, NCH), jnp.float32)

    def tap(t, carry):
        xs = x_ref[0, pl.ds(t + (PAD - 11), L), :]
        acc_ref[...] += xs * w1c_ref[pl.ds(t, 1), :]
        return carry
    lax.fori_loop(0, NTAP, tap, 0)

    x0 = x_ref[0, PAD:PAD + L, :]
    idx = lax.broadcasted_iota(jnp.int32, (L, 1), 0)
    lft = jnp.where(idx == 0, -1e30, x_ref[0, PAD - 1:PAD - 1 + L, :])
    rgt = jnp.where(idx == L - 1, -1e30, x_ref[0, PAD + 1:PAD + 1 + L, :])
    acc_ref[...] += jnp.maximum(jnp.maximum(lft, x0), rgt) * _row(vec_ref, 2)

    z = acc_ref[...] * _row(vec_ref, 0) + _row(vec_ref, 1)
    return jnp.maximum(z, 0.0), x0


def _big_inception(j, wb_ref, wg_ref, wm_ref, vec_ref, bn_row, ybuf, xbuf):
    xb = xbuf[PAD:PAD + L, :]
    zb = jnp.dot(xb, wb_ref[j], preferred_element_type=jnp.float32)
    zbh = zb.astype(jnp.bfloat16)

    ybuf[0:YB, :] = jnp.zeros((YB, GK), jnp.bfloat16)
    ybuf[YB - 7 + L:L + 2 * PAD, :] = jnp.zeros((PAD + 7 + (PAD - YB), GK),
                                                jnp.bfloat16)
    for s in range(8):
        ybuf[YB - s:YB - s + L, NB * s:NB * (s + 1)] = zbh

    acc = jnp.dot(ybuf[8:8 + L, :], wg_ref[j, 0],
                  preferred_element_type=jnp.float32)
    for g in range(1, NG):
        acc += jnp.dot(ybuf[8 * (g + 1):8 * (g + 1) + L, :], wg_ref[j, g],
                       preferred_element_type=jnp.float32)

    m = jnp.maximum(jnp.maximum(xbuf[PAD - 1:PAD - 1 + L, :], xb),
                    xbuf[PAD + 1:PAD + 1 + L, :])
    acc += jnp.dot(m, wm_ref[j], preferred_element_type=jnp.float32)

    return acc * _row(vec_ref, bn_row) + _row(vec_ref, bn_row + 1)


def _inception_time_kernel(x_ref, w1c_ref, wb_ref, wg_ref, wm_ref, res2w_ref,
                           wf_ref, vec_ref, out_ref, ybuf, xbuf, acc_ref):
    xbuf[0:PAD, :] = jnp.zeros((PAD, NCH), jnp.bfloat16)
    xbuf[PAD + L:2 * PAD + L, :] = jnp.zeros((PAD, NCH), jnp.bfloat16)

    z, x0 = _first_inception(x_ref, w1c_ref, vec_ref, acc_ref)
    res1 = x0 * _row(vec_ref, 3) + _row(vec_ref, 4)
    xbuf[PAD:PAD + L, :] = z.astype(jnp.bfloat16)
    a = _big_inception(0, wb_ref, wg_ref, wm_ref, vec_ref, 5, ybuf, xbuf)
    xbuf[PAD:PAD + L, :] = jnp.maximum(a, 0.0).astype(jnp.bfloat16)
    a = _big_inception(1, wb_ref, wg_ref, wm_ref, vec_ref, 7, ybuf, xbuf)
    z = jnp.maximum(jnp.maximum(a, 0.0) + res1, 0.0)

    zh = z.astype(jnp.bfloat16)
    xbuf[PAD:PAD + L, :] = zh
    res2 = jnp.dot(zh, res2w_ref[...], preferred_element_type=jnp.float32)
    res2 += _row(vec_ref, 15)
    a = _big_inception(2, wb_ref, wg_ref, wm_ref, vec_ref, 9, ybuf, xbuf)
    xbuf[PAD:PAD + L, :] = jnp.maximum(a, 0.0).astype(jnp.bfloat16)
    a = _big_inception(3, wb_ref, wg_ref, wm_ref, vec_ref, 11, ybuf, xbuf)
    xbuf[PAD:PAD + L, :] = jnp.maximum(a, 0.0).astype(jnp.bfloat16)
    a = _big_inception(4, wb_ref, wg_ref, wm_ref, vec_ref, 13, ybuf, xbuf)
    z = jnp.maximum(jnp.maximum(a, 0.0) + res2, 0.0)

    pooled = jnp.mean(z, axis=0, keepdims=True)
    logits = jnp.dot(pooled, wf_ref[...], preferred_element_type=jnp.float32)
    logits += _row(vec_ref, 16)
    col = lax.broadcasted_iota(jnp.int32, (1, NCH), 1)
    logits = jnp.where(col < 2, logits, -1e30)
    mx = jnp.max(logits, axis=-1, keepdims=True)
    e = jnp.exp(logits - mx)
    out_ref[0] = e * pl.reciprocal(jnp.sum(e, axis=-1, keepdims=True),
                                   approx=False)


@jax.jit
def _forward(x, w1c, wb_all, wc_all, wm_all, res2w, wf, vecs):
    B = x.shape[0]
    xp = jnp.pad(x.astype(jnp.float32), ((0, 0), (PAD, PAD)))
    xp = xp.reshape(B, L + 2 * PAD, 1)

    wg = jnp.pad(wc_all.reshape(5, NTAP, NB, NCH),
                 ((0, 0), (0, 1), (0, 0), (0, 0)))
    wg = wg.reshape(5, NG, GK, NCH).astype(jnp.bfloat16)
    wbh = wb_all.astype(jnp.bfloat16)
    wmh = wm_all.astype(jnp.bfloat16)
    r2h = res2w.astype(jnp.bfloat16)

    flops_per_sample = (
        5 * (2 * L * NCH * NB + NG * 2 * L * GK * NCH + 2 * L * NCH * NCH)
        + NTAP * 2 * L * NCH + 2 * L * NCH * NCH + 2 * NCH * NCH)
    weight_bytes = (wg.size + wbh.size + wmh.size + r2h.size) * 2 \
        + (w1c.size + wf.size + vecs.size) * 4
    cost = pl.CostEstimate(flops=B * flops_per_sample,
                           transcendentals=B * NCH,
                           bytes_accessed=weight_bytes + int(xp.size) * 4
                           + B * NCH * 4)

    def resident(a):
        n = a.ndim
        return pl.BlockSpec(a.shape, lambda i: (0,) * n)

    out = pl.pallas_call(
        _inception_time_kernel,
        out_shape=jax.ShapeDtypeStruct((B, 1, NCH), jnp.float32),
        grid=(B,),
        in_specs=[pl.BlockSpec((1, L + 2 * PAD, 1), lambda i: (i, 0, 0)),
                  resident(w1c), resident(wbh), resident(wg),
                  resident(wmh), resident(r2h), resident(wf), resident(vecs)],
        out_specs=pl.BlockSpec((1, 1, NCH), lambda i: (i, 0, 0)),
        scratch_shapes=[pltpu.VMEM((L + 2 * PAD, GK), jnp.bfloat16),
                        pltpu.VMEM((L + 2 * PAD, NCH), jnp.bfloat16),
                        pltpu.VMEM((L, NCH), jnp.float32)],
        compiler_params=pltpu.CompilerParams(
            dimension_semantics=("parallel",),
            vmem_limit_bytes=64 << 20),
        cost_estimate=cost,
    )(xp, w1c, wbh, wg, wmh, r2h, wf, vecs)
    return out[:, 0, :2]


def kernel(x, w1c, wb_all, wc_all, wm_all, res2w, wf, vecs):
    return _forward(x, w1c, wb_all, wc_all, wm_all, res2w, wf, vecs)


# speedup vs baseline: 4.6694x; 2.6438x over previous
import jax
import jax.numpy as jnp
from jax import lax
from jax.experimental import pallas as pl
from jax.experimental.pallas import tpu as pltpu

L = 1600
NCH = 128
NB = 32
F = 8
M = L // F
RP = 4
R = M + 2 * RP
FC = F * NCH
FB = F * NB
NQ = 5
KTAP = NQ * FB
KTOT = KTAP + FC
NTAP = 23


def _kernel(x_ref, w1_ref, wr1_ref, wb_ref, wstk_ref, r2_ref, wf_ref, vf_ref,
            out_ref, xact, zbuf, zcat, xcat1):
    zbuf[0:2, :] = jnp.zeros((2, FB), jnp.bfloat16)
    zbuf[R + 2:R + 4, :] = jnp.zeros((2, FB), jnp.bfloat16)
    xcat1[:, NQ * F + F:NCH] = jnp.zeros((R, NCH - NQ * F - F), jnp.float32)

    for qi in range(NQ):
        xcat1[:, F * qi:F * (qi + 1)] = x_ref[0, qi:qi + R, :]
    xv = x_ref[0, 2:2 + R, :]
    lf = jnp.concatenate([x_ref[0, 1:1 + R, 7:8], xv[:, 0:7]], axis=1)
    rf = jnp.concatenate([xv[:, 1:8], x_ref[0, 3:3 + R, 0:1]], axis=1)
    ti = (8 * (lax.broadcasted_iota(jnp.int32, (R, F), 0) - RP)
          + lax.broadcasted_iota(jnp.int32, (R, F), 1))
    lf = jnp.where(ti == 0, -1e30, lf)
    rf = jnp.where(ti == L - 1, -1e30, rf)
    xcat1[:, NQ * F:NQ * F + F] = jnp.maximum(jnp.maximum(lf, xv), rf)
    acc = jnp.dot(xcat1[...], w1_ref[...], preferred_element_type=jnp.float32)
    z = jnp.maximum(acc + vf_ref[1:2, :], 0.0)
    res1 = jnp.dot(xv, wr1_ref[...], preferred_element_type=jnp.float32)
    res1 += vf_ref[4:5, :]

    def store_act(zval):
        xact[...] = zval.astype(jnp.bfloat16)
        xact[0:RP, :] = jnp.zeros((RP, FC), jnp.bfloat16)
        xact[RP + M:R, :] = jnp.zeros((RP, FC), jnp.bfloat16)

    def big(j):
        a = xact[...]
        zf = jnp.dot(a, wb_ref[j], preferred_element_type=jnp.float32)
        zbuf[2:2 + R, :] = zf.astype(jnp.bfloat16)
        for qi in range(NQ):
            zcat[:, FB * qi:FB * (qi + 1)] = zbuf[qi:qi + R, :]
        zb16 = jnp.zeros((1, NCH), jnp.bfloat16)
        lfa = jnp.concatenate(
            [jnp.concatenate([zb16, a[0:R - 1, FC - NCH:FC]], axis=0),
             a[:, 0:FC - NCH]], axis=1)
        rfa = jnp.concatenate(
            [a[:, NCH:FC],
             jnp.concatenate([a[1:R, 0:NCH], zb16], axis=0)], axis=1)
        zcat[:, KTAP:KTOT] = jnp.maximum(jnp.maximum(lfa, a), rfa)
        return jnp.dot(zcat[...], wstk_ref[j],
                       preferred_element_type=jnp.float32)

    store_act(z)
    a = big(0)
    store_act(jnp.maximum(a + vf_ref[6:7, :], 0.0))
    a = big(1)
    z = jnp.maximum(jnp.maximum(a + vf_ref[8:9, :], 0.0) + res1, 0.0)

    zb = z.astype(jnp.bfloat16)
    xact[...] = zb
    xact[0:RP, :] = jnp.zeros((RP, FC), jnp.bfloat16)
    xact[RP + M:R, :] = jnp.zeros((RP, FC), jnp.bfloat16)
    res2 = jnp.dot(zb, r2_ref[...], preferred_element_type=jnp.float32)
    res2 += vf_ref[15:16, :]
    a = big(2)
    store_act(jnp.maximum(a + vf_ref[10:11, :], 0.0))
    a = big(3)
    store_act(jnp.maximum(a + vf_ref[12:13, :], 0.0))
    a = big(4)
    z = jnp.maximum(jnp.maximum(a + vf_ref[14:15, :], 0.0) + res2, 0.0)

    ki = lax.broadcasted_iota(jnp.int32, (R, 1), 0)
    zm = jnp.where((ki >= RP) & (ki < RP + M), z, 0.0)
    s1 = jnp.sum(zm, axis=0, keepdims=True)
    pooled = s1[:, 0:NCH]
    for p in range(1, F):
        pooled = pooled + s1[:, NCH * p:NCH * (p + 1)]
    pooled = pooled * (1.0 / L)
    logits = jnp.dot(pooled, wf_ref[...], preferred_element_type=jnp.float32)
    logits += vf_ref[16:17, 0:NCH]
    col = lax.broadcasted_iota(jnp.int32, (1, NCH), 1)
    logits = jnp.where(col < 2, logits, -1e30)
    mx = jnp.max(logits, axis=-1, keepdims=True)
    e = jnp.exp(logits - mx)
    out_ref[0] = e * pl.reciprocal(jnp.sum(e, axis=-1, keepdims=True),
                                   approx=False)


@jax.jit
def _forward(x, w1c, wb_all, wc_all, wm_all, res2w, wf, vecs):
    B = x.shape[0]
    f32 = jnp.float32
    bf16 = jnp.bfloat16
    xf = jnp.pad(x.astype(f32).reshape(B, M, F), ((0, 0), (RP + 2, RP + 2),
                                                  (0, 0)))

    qs = jnp.arange(NQ)[:, None, None]
    ss = jnp.arange(F)[None, :, None]
    ps = jnp.arange(F)[None, None, :]
    t = 8 * (qs - 2) + ss - ps + 11
    valid = (t >= 0) & (t < NTAP)
    tc = jnp.clip(t, 0, NTAP - 1)
    eye8 = jnp.eye(F, dtype=f32)
    scale = vecs[jnp.array([5, 7, 9, 11, 13])]

    wc5 = wc_all.reshape(5, NTAP, NB, NCH)
    wtap = wc5[:, tc]
    wtap = jnp.where(valid[None, :, :, :, None, None], wtap, 0.0)
    wtap = wtap.transpose(0, 1, 2, 4, 3, 5)
    wtap = wtap * scale[:, None, None, None, None, :]
    wtap = wtap.reshape(5, KTAP, FC)
    wmp = jnp.einsum('ps,jco->jpcso', eye8, wm_all)
    wmp = wmp * scale[:, None, None, None, :]
    wstk = jnp.concatenate([wtap, wmp.reshape(5, FC, FC)],
                           axis=1).astype(bf16)
    wb_bd = jnp.einsum('ps,jco->jpcso', eye8, wb_all)
    wb_bd = wb_bd.reshape(5, FC, FB).astype(bf16)
    r2bd = jnp.einsum('ps,co->pcso', eye8, res2w).reshape(FC, FC).astype(bf16)

    w1s = w1c * vecs[0][None, :]
    w1tap = jnp.where(valid[..., None], w1s[tc], 0.0)
    w1mp = jnp.einsum('ps,c->psc', eye8, vecs[2] * vecs[0])
    w1 = jnp.concatenate([w1tap.reshape(NQ * F, FC),
                          w1mp.reshape(F, FC),
                          jnp.zeros((NCH - NQ * F - F, FC), f32)], axis=0)
    wr1 = jnp.einsum('ps,c->psc', eye8, vecs[3]).reshape(F, FC)
    vf = jnp.tile(vecs, (1, F))

    flops_per_sample = (
        5 * (2 * L * FC // F * NB * F + 2 * L * KTOT * NCH)
        + 2 * L * NCH + 2 * L * NCH * NCH + 2 * NCH * NCH)
    weight_bytes = (wstk.size + wb_bd.size + r2bd.size) * 2 \
        + (w1.size + wr1.size + wf.size + vf.size) * 4
    cost = pl.CostEstimate(flops=B * flops_per_sample,
                           transcendentals=B * NCH,
                           bytes_accessed=weight_bytes + int(xf.size) * 4
                           + B * NCH * 4)

    def resident(a):
        n = a.ndim
        return pl.BlockSpec(a.shape, lambda i: (0,) * n)

    out = pl.pallas_call(
        _kernel,
        out_shape=jax.ShapeDtypeStruct((B, 1, NCH), f32),
        grid=(B,),
        in_specs=[pl.BlockSpec((1, R + 4, F), lambda i: (i, 0, 0)),
                  resident(w1), resident(wr1), resident(wb_bd),
                  resident(wstk), resident(r2bd), resident(wf), resident(vf)],
        out_specs=pl.BlockSpec((1, 1, NCH), lambda i: (i, 0, 0)),
        scratch_shapes=[pltpu.VMEM((R, FC), bf16),
                        pltpu.VMEM((R + 4, FB), bf16),
                        pltpu.VMEM((R, KTOT), bf16),
                        pltpu.VMEM((R, NCH), f32)],
        compiler_params=pltpu.CompilerParams(
            dimension_semantics=("parallel",),
            vmem_limit_bytes=56 << 20),
        cost_estimate=cost,
    )(xf, w1, wr1, wb_bd, wstk, r2bd, wf, vf)
    return out[:, 0, :2]


def kernel(x, w1c, wb_all, wc_all, wm_all, res2w, wf, vecs):
    return _forward(x, w1c, wb_all, wc_all, wm_all, res2w, wf, vecs)


# speedup vs baseline: 5.5171x; 1.1815x over previous
import jax
import jax.numpy as jnp
from jax import lax
from jax.experimental import pallas as pl
from jax.experimental.pallas import tpu as pltpu

L = 1600
NCH = 128
NB = 32
F = 8
M = L // F
RP = 4
R = M + 2 * RP
FC = F * NCH
FB = F * NB
NQ = 5
KTAP = NQ * FB
KTOT = KTAP + FC
NTAP = 23
BB = 4
RS = BB * R


def _kernel(x_ref, w1_ref, wr1_ref, wb_ref, wstk_ref, r2_ref, wf_ref, vf_ref,
            out_ref, xact, zbuf, zcat, xcat1):
    zbuf[0:2, :] = jnp.zeros((2, FB), jnp.bfloat16)
    zbuf[RS + 2:RS + 4, :] = jnp.zeros((2, FB), jnp.bfloat16)
    xcat1[:, NQ * F + F:NCH] = jnp.zeros((RS, NCH - NQ * F - F), jnp.float32)

    for b in range(BB):
        for qi in range(NQ):
            xcat1[b * R:(b + 1) * R, F * qi:F * (qi + 1)] = \
                x_ref[b, qi:qi + R, :]
    xv = xcat1[:, 16:24]
    lf = jnp.concatenate([xcat1[:, 15:16], xcat1[:, 16:23]], axis=1)
    rf = jnp.concatenate([xcat1[:, 17:24], xcat1[:, 24:25]], axis=1)
    ki = lax.broadcasted_iota(jnp.int32, (RS, F), 0)
    ki = ki - R * (ki // R)
    ti = 8 * (ki - RP) + lax.broadcasted_iota(jnp.int32, (RS, F), 1)
    lf = jnp.where(ti == 0, -1e30, lf)
    rf = jnp.where(ti == L - 1, -1e30, rf)
    xcat1[:, NQ * F:NQ * F + F] = jnp.maximum(jnp.maximum(lf, xv), rf)
    acc = jnp.dot(xcat1[...], w1_ref[...], preferred_element_type=jnp.float32)
    z = jnp.maximum(acc + vf_ref[1:2, :], 0.0)
    res1 = jnp.dot(xv, wr1_ref[...], preferred_element_type=jnp.float32)
    res1 += vf_ref[4:5, :]

    def store_act(zval):
        xact[...] = zval.astype(jnp.bfloat16)
        for b in range(BB):
            xact[b * R:b * R + RP, :] = jnp.zeros((RP, FC), jnp.bfloat16)
            xact[b * R + RP + M:(b + 1) * R, :] = \
                jnp.zeros((RP, FC), jnp.bfloat16)

    def big(j):
        a = xact[...]
        zf = jnp.dot(a, wb_ref[j], preferred_element_type=jnp.float32)
        zbuf[2:2 + RS, :] = zf.astype(jnp.bfloat16)
        for qi in range(NQ):
            zcat[:, FB * qi:FB * (qi + 1)] = zbuf[qi:qi + RS, :]
        zb16 = jnp.zeros((1, NCH), jnp.bfloat16)
        lfa = jnp.concatenate(
            [jnp.concatenate([zb16, a[0:RS - 1, FC - NCH:FC]], axis=0),
             a[:, 0:FC - NCH]], axis=1)
        rfa = jnp.concatenate(
            [a[:, NCH:FC],
             jnp.concatenate([a[1:RS, 0:NCH], zb16], axis=0)], axis=1)
        zcat[:, KTAP:KTOT] = jnp.maximum(jnp.maximum(lfa, a), rfa)
        return jnp.dot(zcat[...], wstk_ref[j],
                       preferred_element_type=jnp.float32)

    store_act(z)
    a = big(0)
    store_act(jnp.maximum(a + vf_ref[6:7, :], 0.0))
    a = big(1)
    z = jnp.maximum(jnp.maximum(a + vf_ref[8:9, :], 0.0) + res1, 0.0)

    store_act(z)
    res2 = jnp.dot(z.astype(jnp.bfloat16), r2_ref[...],
                   preferred_element_type=jnp.float32)
    res2 += vf_ref[15:16, :]
    a = big(2)
    store_act(jnp.maximum(a + vf_ref[10:11, :], 0.0))
    a = big(3)
    store_act(jnp.maximum(a + vf_ref[12:13, :], 0.0))
    a = big(4)
    z = jnp.maximum(jnp.maximum(a + vf_ref[14:15, :], 0.0) + res2, 0.0)

    kr = lax.broadcasted_iota(jnp.int32, (RS, 1), 0)
    kr = kr - R * (kr // R)
    zm = jnp.where((kr >= RP) & (kr < RP + M), z, 0.0)
    pooled = []
    for b in range(BB):
        s1 = jnp.sum(zm[b * R:(b + 1) * R], axis=0, keepdims=True)
        p1 = s1[:, 0:NCH]
        for p in range(1, F):
            p1 = p1 + s1[:, NCH * p:NCH * (p + 1)]
        pooled.append(p1)
    pooled = jnp.concatenate(pooled, axis=0) * (1.0 / L)
    logits = jnp.dot(pooled, wf_ref[...], preferred_element_type=jnp.float32)
    logits += vf_ref[16:17, 0:NCH]
    col = lax.broadcasted_iota(jnp.int32, (BB, NCH), 1)
    logits = jnp.where(col < 2, logits, -1e30)
    mx = jnp.max(logits, axis=-1, keepdims=True)
    e = jnp.exp(logits - mx)
    out_ref[:, 0, :] = e * pl.reciprocal(jnp.sum(e, axis=-1, keepdims=True),
                                         approx=False)


@jax.jit
def _forward(x, w1c, wb_all, wc_all, wm_all, res2w, wf, vecs):
    B = x.shape[0]
    f32 = jnp.float32
    bf16 = jnp.bfloat16
    xf = jnp.pad(x.astype(f32).reshape(B, M, F), ((0, 0), (RP + 2, RP + 2),
                                                  (0, 0)))

    qs = jnp.arange(NQ)[:, None, None]
    ss = jnp.arange(F)[None, :, None]
    ps = jnp.arange(F)[None, None, :]
    t = 8 * (qs - 2) + ss - ps + 11
    valid = (t >= 0) & (t < NTAP)
    tc = jnp.clip(t, 0, NTAP - 1)
    eye8 = jnp.eye(F, dtype=f32)
    scale = vecs[jnp.array([5, 7, 9, 11, 13])]

    wc5 = wc_all.reshape(5, NTAP, NB, NCH)
    wtap = wc5[:, tc]
    wtap = jnp.where(valid[None, :, :, :, None, None], wtap, 0.0)
    wtap = wtap.transpose(0, 1, 2, 4, 3, 5)
    wtap = wtap * scale[:, None, None, None, None, :]
    wtap = wtap.reshape(5, KTAP, FC)
    wmp = jnp.einsum('ps,jco->jpcso', eye8, wm_all)
    wmp = wmp * scale[:, None, None, None, :]
    wstk = jnp.concatenate([wtap, wmp.reshape(5, FC, FC)],
                           axis=1).astype(bf16)
    wb_bd = jnp.einsum('ps,jco->jpcso', eye8, wb_all)
    wb_bd = wb_bd.reshape(5, FC, FB).astype(bf16)
    r2bd = jnp.einsum('ps,co->pcso', eye8, res2w).reshape(FC, FC).astype(bf16)

    w1s = w1c * vecs[0][None, :]
    w1tap = jnp.where(valid[..., None], w1s[tc], 0.0)
    w1mp = jnp.einsum('ps,c->psc', eye8, vecs[2] * vecs[0])
    w1 = jnp.concatenate([w1tap.reshape(NQ * F, FC),
                          w1mp.reshape(F, FC),
                          jnp.zeros((NCH - NQ * F - F, FC), f32)], axis=0)
    wr1 = jnp.einsum('ps,c->psc', eye8, vecs[3]).reshape(F, FC)
    vf = jnp.tile(vecs, (1, F))

    flops_per_sample = (
        5 * (2 * L * NCH * NB + 2 * L * KTOT * NCH)
        + 2 * L * NCH + 2 * L * NCH * NCH + 2 * NCH * NCH)
    weight_bytes = (wstk.size + wb_bd.size + r2bd.size) * 2 \
        + (w1.size + wr1.size + wf.size + vf.size) * 4
    cost = pl.CostEstimate(flops=B * flops_per_sample,
                           transcendentals=B * NCH,
                           bytes_accessed=weight_bytes + int(xf.size) * 4
                           + B * NCH * 4)

    def resident(a):
        n = a.ndim
        return pl.BlockSpec(a.shape, lambda i: (0,) * n)

    out = pl.pallas_call(
        _kernel,
        out_shape=jax.ShapeDtypeStruct((B, 1, NCH), f32),
        grid=(B // BB,),
        in_specs=[pl.BlockSpec((BB, R + 4, F), lambda i: (i, 0, 0)),
                  resident(w1), resident(wr1), resident(wb_bd),
                  resident(wstk), resident(r2bd), resident(wf), resident(vf)],
        out_specs=pl.BlockSpec((BB, 1, NCH), lambda i: (i, 0, 0)),
        scratch_shapes=[pltpu.VMEM((RS, FC), bf16),
                        pltpu.VMEM((RS + 4, FB), bf16),
                        pltpu.VMEM((RS, KTOT), bf16),
                        pltpu.VMEM((RS, NCH), f32)],
        compiler_params=pltpu.CompilerParams(
            dimension_semantics=("parallel",),
            vmem_limit_bytes=100 << 20),
        cost_estimate=cost,
    )(xf, w1, wr1, wb_bd, wstk, r2bd, wf, vf)
    return out[:, 0, :2]


def kernel(x, w1c, wb_all, wc_all, wm_all, res2w, wf, vecs):
    return _forward(x, w1c, wb_all, wc_all, wm_all, res2w, wf, vecs)
